# Initial kernel scaffold; baseline (speedup 1.0000x reference)
import jax
import jax.numpy as jnp
from jax.experimental import pallas as pl


def kernel(h, x, edge_index, edge_attr, batch, params):
    raise NotImplementedError("write your pallas kernel here")


# speedup vs baseline: 1.9735x; 1.9735x over previous
import functools

import jax
import jax.numpy as jnp
from jax import lax
from jax.experimental import pallas as pl
from jax.experimental.pallas import tpu as pltpu
from jax.experimental.pallas import tpu_sc as plsc

N = 10000
E = 320000
B = 64
H = 128
L = 4

NPAD = 10240
NCORE = 2
NSUB = 16
CHUNK = 128
CHUNKS_PER_TILE = 79
PER_TILE = CHUNK * CHUNKS_PER_TILE
EPAD = NCORE * NSUB * PER_TILE
TW = 256
SW = 128
EBLK = 1024
NBLK = 1024
NPB = NPAD // NBLK

_f32 = jnp.float32


def _silu(t):
    return t * jax.nn.sigmoid(t)


def _ln(t, g, b):
    mu = jnp.mean(t, -1, keepdims=True)
    d = t - mu
    var = jnp.mean(d * d, -1, keepdims=True)
    return d / jnp.sqrt(var + 1e-5) * g + b


def _wspec(shape):
    nd = len(shape)
    return pl.BlockSpec(shape, lambda i: (0,) * nd)


def _bspec(shape):
    return pl.BlockSpec(shape, lambda i: (i,) + (0,) * (len(shape) - 1))


def _tables(h, x16, wa, wb):
    z = jnp.zeros((h.shape[0], TW - 144), _f32)
    a = jnp.concatenate(
        [jnp.dot(h, wa[...], preferred_element_type=_f32), x16, z], axis=1)
    b = jnp.concatenate(
        [jnp.dot(h, wb[...], preferred_element_type=_f32), x16, z], axis=1)
    return a, b


def _enc_body(hin, x16, new, neb, neg, nebeta, wa, wb, h_out, a_out, b_out):
    hp = _silu(jnp.dot(hin[...], new[...], preferred_element_type=_f32) + neb[...])
    h = _ln(hp, neg[...], nebeta[...])
    h_out[...] = h
    a_out[...], b_out[...] = _tables(h, x16[...], wa, wb)


def _edge_enc_body(eap, eew, eeb, out):
    out[...] = _silu(jnp.dot(eap[...], eew[...], preferred_element_type=_f32)
                     + eeb[...])


def _edge_body(has_coord, av, bv, ea, wd, wc, eb1, ew2, eb2, cw1, cb1, cw2,
               m_out, t_out=None):
    a = av[...]
    b = bv[...]
    hsum = a[:, 0:128] + b[:, 0:128]
    xd16 = a[:, 128:144] - b[:, 128:144]
    dist_sq = jnp.sum(xd16 * xd16, axis=1, keepdims=True)
    e1 = (hsum
          + jnp.dot(ea[...], wd[...], preferred_element_type=_f32)
          + dist_sq * wc[...] + eb1[...])
    m = _silu(jnp.dot(_silu(e1), ew2[...], preferred_element_type=_f32) + eb2[...])
    m_out[...] = m
    if has_coord:
        lane = lax.broadcasted_iota(jnp.int32, (EBLK, 16), 1)
        t = _silu(jnp.dot(m, cw1[...], preferred_element_type=_f32) + cb1[...])
        c = jnp.tanh(jnp.dot(t, cw2[...], preferred_element_type=_f32))
        dist = jnp.sqrt(dist_sq + 1e-8)
        tail = xd16 * (c / dist)
        tail = jnp.where(lane == 3, 1.0, tail)
        t_out[...] = jnp.concatenate(
            [tail, jnp.zeros((EBLK, SW - 16), _f32)], axis=1)


def _node_body(has_coord, *refs):
    if has_coord:
        (h_in, x16, p0, p1, t0, t1, nw1h, nw1m, nb1, nw2, nb2, lng, lnb,
         wa, wb, h_out, x_out, a_out, b_out) = refs
    else:
        h_in, p0, p1, nw1h, nw1m, nb1, nw2, nb2, lng, lnb, h_out = refs
    h = h_in[...]
    m_i = p0[:, 0:128] + p1[:, 0:128]
    hu = _silu(jnp.dot(h, nw1h[...], preferred_element_type=_f32)
               + jnp.dot(m_i, nw1m[...], preferred_element_type=_f32)
               + nb1[...])
    hu = jnp.dot(hu, nw2[...], preferred_element_type=_f32) + nb2[...]
    hn = _ln(h + hu, lng[...], lnb[...])
    h_out[...] = hn
    if has_coord:
        tail = t0[:, 0:16] + t1[:, 0:16]
        deg = jnp.maximum(tail[:, 3:4], 1.0)
        lane = lax.broadcasted_iota(jnp.int32, (NBLK, 16), 1)
        xn = x16[...] + jnp.where(lane < 3, tail, 0.0) / deg
        x_out[...] = xn
        a_out[...], b_out[...] = _tables(hn, xn, wa, wb)


def _pool_body(h_ref, bp_ref, pw1, pb1, pw2, pb2, cw1, cb1, cw2, cb2, cw3, cb3,
               out_ref):
    h = h_ref[...]
    bp = bp_ref[...]
    s = jnp.dot(jnp.tanh(jnp.dot(h, pw1[...], preferred_element_type=_f32)
                         + pb1[...]),
                pw2[...], preferred_element_type=_f32) + pb2[...]
    cols = lax.broadcasted_iota(jnp.int32, (NPAD, B), 1)
    m = bp == cols
    mf = m.astype(_f32)
    dn = (((0,), (0,)), ((), ()))
    smax = jnp.max(jnp.where(m, s, -1e30), axis=0, keepdims=True)
    sg = jnp.sum(jnp.where(m, smax, 0.0), axis=1, keepdims=True)
    sexp = jnp.exp(s - sg)
    ssum = lax.dot_general(sexp, mf, dn, preferred_element_type=_f32)
    sden = jnp.sum(jnp.where(m, ssum, 0.0), axis=1, keepdims=True)
    w = jnp.where(bp >= 0, sexp / (sden + 1e-16), 0.0)
    g = lax.dot_general(mf, h * w, dn, preferred_element_type=_f32)
    inv = 0.9999950000374996
    z = _silu(jnp.dot(g, cw1[...], preferred_element_type=_f32) + cb1[...]) * inv
    z = _silu(jnp.dot(z, cw2[...], preferred_element_type=_f32) + cb2[...]) * inv
    out_ref[...] = jnp.dot(z, cw3[...], preferred_element_type=_f32) + cb3[...]


def _gather_body(tab_hbm, idx_hbm, out_hbm, idx_v, buf_v, sem):
    wid = lax.axis_index("c") * NSUB + lax.axis_index("s")
    base = wid * PER_TILE

    @pl.loop(0, CHUNKS_PER_TILE)
    def _chunk(t):
        off = base + t * CHUNK
        pltpu.sync_copy(idx_hbm.at[pl.ds(off, CHUNK)], idx_v)
        pltpu.async_copy(tab_hbm.at[idx_v], buf_v, sem).wait()
        pltpu.sync_copy(buf_v, out_hbm.at[pl.ds(off, CHUNK)])


_gather = pl.kernel(
    _gather_body,
    out_type=jax.ShapeDtypeStruct((EPAD, TW), _f32),
    mesh=plsc.VectorSubcoreMesh(core_axis_name="c", subcore_axis_name="s"),
    scratch_types=[
        pltpu.VMEM((CHUNK,), jnp.int32),
        pltpu.VMEM((CHUNK, TW), _f32),
        pltpu.SemaphoreType.DMA,
    ],
)


def _scatter_body(row_hbm, mv_hbm, zero_hbm, out_hbm, idx_v, vals_v, acc_sh):
    cid = lax.axis_index("c")
    sid = lax.axis_index("s")
    wid = cid * NSUB + sid
    rows = NPAD // NSUB
    rbase = sid * rows
    pltpu.sync_copy(zero_hbm.at[pl.ds(rbase, rows)],
                    acc_sh.at[pl.ds(rbase, rows)])
    plsc.subcore_barrier()
    base = wid * PER_TILE

    @pl.loop(0, CHUNKS_PER_TILE)
    def _chunk(t):
        off = base + t * CHUNK
        pltpu.sync_copy(row_hbm.at[pl.ds(off, CHUNK)], idx_v)
        pltpu.sync_copy(mv_hbm.at[pl.ds(off, CHUNK)], vals_v)
        pltpu.sync_copy(vals_v, acc_sh.at[idx_v], add=True)

    plsc.subcore_barrier()
    obase = cid * NPAD + rbase
    pltpu.sync_copy(acc_sh.at[pl.ds(rbase, rows)],
                    out_hbm.at[pl.ds(obase, rows)])


_scatter = pl.kernel(
    _scatter_body,
    out_type=jax.ShapeDtypeStruct((NCORE * NPAD, SW), _f32),
    mesh=plsc.VectorSubcoreMesh(core_axis_name="c", subcore_axis_name="s"),
    scratch_types=[
        pltpu.VMEM((CHUNK,), jnp.int32),
        pltpu.VMEM((CHUNK, SW), _f32),
        pltpu.VMEM_SHARED((NPAD, SW), _f32),
    ],
)


def kernel(h, x, edge_index, edge_attr, batch, params):
    p = params
    r2 = lambda t: t.reshape(1, -1)

    hp = jnp.pad(h, ((0, NPAD - N), (0, 64 - 58)))
    x16 = jnp.pad(x, ((0, NPAD - N), (0, 13)))
    eap = jnp.pad(edge_attr, ((0, EPAD - E), (0, 4)))
    rowp = jnp.pad(edge_index[0], (0, EPAD - E), constant_values=NPAD - 1)
    colp = jnp.pad(edge_index[1], (0, EPAD - E), constant_values=NPAD - 1)
    bp = jnp.pad(batch, (0, NPAD - N), constant_values=-1).reshape(NPAD, 1)
    zeros_acc = jnp.zeros((NPAD, SW), _f32)
    new_p = jnp.pad(p['ne_w'], ((0, 6), (0, 0)))
    eew_p = jnp.pad(p['ee_w'], ((0, 4), (0, 0)))

    lw = p['layers']
    wa0 = lw[0]['ew1'][0:128]
    wb0 = lw[0]['ew1'][128:256]

    grid_n = (NPB,)
    grid_e = (EPAD // EBLK,)

    hcur, A, Bt = pl.pallas_call(
        _enc_body,
        grid=grid_n,
        in_specs=[
            _bspec((NBLK, 64)), _bspec((NBLK, 16)),
            _wspec((64, 128)), _wspec((1, 128)), _wspec((1, 128)),
            _wspec((1, 128)), _wspec((128, 128)), _wspec((128, 128)),
        ],
        out_specs=[_bspec((NBLK, 128)), _bspec((NBLK, TW)), _bspec((NBLK, TW))],
        out_shape=[
            jax.ShapeDtypeStruct((NPAD, 128), _f32),
            jax.ShapeDtypeStruct((NPAD, TW), _f32),
            jax.ShapeDtypeStruct((NPAD, TW), _f32),
        ],
    )(hp, x16, new_p, r2(p['ne_b']), r2(p['ne_g']), r2(p['ne_beta']), wa0, wb0)

    ea = pl.pallas_call(
        _edge_enc_body,
        grid=grid_e,
        in_specs=[_bspec((EBLK, 16)), _wspec((16, 128)), _wspec((1, 128))],
        out_specs=_bspec((EBLK, 128)),
        out_shape=jax.ShapeDtypeStruct((EPAD, 128), _f32),
    )(eap, eew_p, r2(p['ee_b']))

    for i in range(L):
        lp = lw[i]
        has_coord = i < L - 1
        wc = lp['ew1'][256:257]
        wd = lp['ew1'][257:385]

        av = _gather(A, rowp)
        bv = _gather(Bt, colp)

        if has_coord:
            cw1, cb1, cw2 = lp['cw1'], r2(lp['cb1']), lp['cw2']
            n_out = 2
        else:
            cw1 = jnp.zeros((128, 128), _f32)
            cb1 = jnp.zeros((1, 128), _f32)
            cw2 = jnp.zeros((128, 1), _f32)
            n_out = 1
        eouts = pl.pallas_call(
            functools.partial(_edge_body, has_coord),
            grid=grid_e,
            in_specs=[
                _bspec((EBLK, TW)), _bspec((EBLK, TW)), _bspec((EBLK, 128)),
                _wspec((128, 128)), _wspec((1, 128)), _wspec((1, 128)),
                _wspec((128, 128)), _wspec((1, 128)),
                _wspec((128, 128)), _wspec((1, 128)), _wspec((128, 1)),
            ],
            out_specs=[_bspec((EBLK, SW))] * n_out,
            out_shape=[jax.ShapeDtypeStruct((EPAD, SW), _f32)] * n_out,
        )(av, bv, ea, wd, wc, r2(lp['eb1']), lp['ew2'], r2(lp['eb2']),
          cw1, cb1, cw2)
        if has_coord:
            mv, tv = eouts
        else:
            mv, = eouts

        parts_m = _scatter(rowp, mv, zeros_acc)
        if has_coord:
            parts_t = _scatter(rowp, tv, zeros_acc)

        nw1h = lp['nw1'][0:128]
        nw1m = lp['nw1'][128:256]
        if has_coord:
            wan = lw[i + 1]['ew1'][0:128]
            wbn = lw[i + 1]['ew1'][128:256]
            hcur, x16, A, Bt = pl.pallas_call(
                functools.partial(_node_body, True),
                grid=grid_n,
                in_specs=[
                    _bspec((NBLK, 128)), _bspec((NBLK, 16)),
                    _bspec((NBLK, SW)),
                    pl.BlockSpec((NBLK, SW), lambda j: (NPB + j, 0)),
                    _bspec((NBLK, SW)),
                    pl.BlockSpec((NBLK, SW), lambda j: (NPB + j, 0)),
                    _wspec((128, 128)), _wspec((128, 128)), _wspec((1, 128)),
                    _wspec((128, 128)), _wspec((1, 128)),
                    _wspec((1, 128)), _wspec((1, 128)),
                    _wspec((128, 128)), _wspec((128, 128)),
                ],
                out_specs=[_bspec((NBLK, 128)), _bspec((NBLK, 16)),
                           _bspec((NBLK, TW)), _bspec((NBLK, TW))],
                out_shape=[
                    jax.ShapeDtypeStruct((NPAD, 128), _f32),
                    jax.ShapeDtypeStruct((NPAD, 16), _f32),
                    jax.ShapeDtypeStruct((NPAD, TW), _f32),
                    jax.ShapeDtypeStruct((NPAD, TW), _f32),
                ],
            )(hcur, x16, parts_m, parts_m, parts_t, parts_t,
              nw1h, nw1m, r2(lp['nb1']), lp['nw2'],
              r2(lp['nb2']), r2(lp['ln_g']), r2(lp['ln_b']), wan, wbn)
        else:
            hcur = pl.pallas_call(
                functools.partial(_node_body, False),
                grid=grid_n,
                in_specs=[
                    _bspec((NBLK, 128)),
                    _bspec((NBLK, SW)),
                    pl.BlockSpec((NBLK, SW), lambda j: (NPB + j, 0)),
                    _wspec((128, 128)), _wspec((128, 128)), _wspec((1, 128)),
                    _wspec((128, 128)), _wspec((1, 128)),
                    _wspec((1, 128)), _wspec((1, 128)),
                ],
                out_specs=_bspec((NBLK, 128)),
                out_shape=jax.ShapeDtypeStruct((NPAD, 128), _f32),
            )(hcur, parts_m, parts_m, nw1h, nw1m, r2(lp['nb1']), lp['nw2'],
              r2(lp['nb2']), r2(lp['ln_g']), r2(lp['ln_b']))

    out = pl.pallas_call(
        _pool_body,
        out_shape=jax.ShapeDtypeStruct((B, 1), _f32),
    )(hcur, bp, p['pw1'], r2(p['pb1']), p['pw2'], r2(p['pb2']),
      p['cw1'], r2(p['cb1']), p['cw2'], r2(p['cb2']), p['cw3'], r2(p['cb3']))
    return out
